# baseline (speedup 1.0000x reference)
import functools

import jax
import jax.numpy as jnp
from jax.experimental import pallas as pl
from jax.experimental.pallas import tpu as pltpu
from jax.experimental.pallas import tpu_sc as plsc

_BATCH = 1024
_ENT = 64
_DIM = 256
_CURR = 512
_RB = 64
_RF = _RB * _ENT


def _fused_body(starts_ref, order_ref, rows_ref,
                h_ref, k_ref, s_ref, u_ref, v_ref, w_ref,
                o_ref,
                sw_ref, sb_ref, uvb_ref, acc_ref, hkb_ref, m_ref):
    i = pl.program_id(0)

    @pl.when(i == 0)
    def _():
        uvb_ref[...] = (u_ref[...] + v_ref[...]).astype(jnp.bfloat16)
        sb_ref[0:_CURR, :] = s_ref[...]
        sb_ref[_CURR:, :] = jnp.zeros((8, _DIM), jnp.float32)
        sw_ref[...] = jnp.dot(s_ref[...], w_ref[...],
                              preferred_element_type=jnp.float32)

    hf = h_ref[...].reshape(_RF, _DIM)
    acc_ref[...] = hf
    hfb = hf.astype(jnp.bfloat16)
    hkb_ref[...] = hfb + k_ref[...].reshape(_RF, _DIM).astype(jnp.bfloat16)
    m_ref[...] = jnp.dot(hfb, uvb_ref[...],
                         preferred_element_type=jnp.float32)

    lo = starts_ref[i * _RB]
    hi = starts_ref[i * _RB + _RB]

    def seg_body(j, carry):
        c = order_ref[j]
        off = (rows_ref[j] - i * _RB) * _ENT
        s_c = sb_ref[pl.ds(c, 1), :]
        hk = hkb_ref[pl.ds(off, _ENT), :].astype(jnp.float32)
        gate = jax.nn.sigmoid(jax.lax.dot_general(
            hk, s_c, (((1,), (1,)), ((), ())),
            preferred_element_type=jnp.float32))
        sw = sw_ref[pl.ds((c % 8) * _ENT, _ENT), :]
        m = m_ref[pl.ds(off, _ENT), :]
        acc_ref[pl.ds(off, _ENT), :] += gate * jnp.maximum(m + sw, 0.0)
        return carry

    jax.lax.fori_loop(lo, hi, seg_body, 0, unroll=False)

    a = acc_ref[...]
    sq = jnp.sum(a * a, axis=1, keepdims=True)
    o_ref[...] = (a * jax.lax.rsqrt(jnp.maximum(sq, 1e-12))).reshape(
        _RB, _ENT, _DIM)


def _route_sc(indices):
    out_type = (jax.ShapeDtypeStruct((_BATCH + 1,), jnp.int32),
                jax.ShapeDtypeStruct((_CURR,), jnp.int32),
                jax.ShapeDtypeStruct((_CURR,), jnp.int32))
    mesh = plsc.ScalarSubcoreMesh(axis_name="core", num_cores=2)

    @functools.partial(
        pl.kernel, out_type=out_type, mesh=mesh,
        scratch_types=[pltpu.SMEM((_CURR,), jnp.int32),
                       pltpu.SMEM((_BATCH + 1,), jnp.int32),
                       pltpu.SMEM((_BATCH,), jnp.int32),
                       pltpu.SMEM((_CURR,), jnp.int32),
                       pltpu.SMEM((_CURR,), jnp.int32),
                       pltpu.SemaphoreType.DMA])
    def route_kernel(idx_hbm, starts_hbm, order_hbm, rows_hbm,
                     idx_s, starts_s, cur_s, order_s, rows_s, sem):
        @pl.when(jax.lax.axis_index("core") == 0)
        def _():
            pltpu.async_copy(idx_hbm, idx_s, sem).wait()

            @pl.loop(0, _BATCH + 1)
            def _(b):
                starts_s[b] = 0

            @pl.loop(0, _CURR)
            def _(c):
                starts_s[idx_s[c] + 1] += 1

            @pl.loop(0, _BATCH)
            def _(b):
                starts_s[b + 1] += starts_s[b]
                cur_s[b] = starts_s[b]

            @pl.loop(0, _CURR)
            def _(c):
                v = idx_s[c]
                p = cur_s[v]
                order_s[p] = c
                rows_s[p] = v
                cur_s[v] = p + 1

            pltpu.async_copy(starts_s, starts_hbm, sem).wait()
            pltpu.async_copy(order_s, order_hbm, sem).wait()
            pltpu.async_copy(rows_s, rows_hbm, sem).wait()

    return route_kernel(indices)


@functools.partial(jax.jit, static_argnames=("interpret",))
def _run(encoded_sents, indices, hiddens, keys, U, V, W, interpret=False):
    indices = indices.astype(jnp.int32)
    if interpret:
        counts = jnp.zeros((_BATCH,), jnp.int32).at[indices].add(1)
        starts = jnp.concatenate(
            [jnp.zeros((1,), jnp.int32), jnp.cumsum(counts, dtype=jnp.int32)])
        order = jnp.argsort(indices).astype(jnp.int32)
        rows = indices[order]
    else:
        starts, order, rows = _route_sc(indices)

    grid_spec = pltpu.PrefetchScalarGridSpec(
        num_scalar_prefetch=3,
        grid=(_BATCH // _RB,),
        in_specs=[
            pl.BlockSpec((_RB, _ENT, _DIM), lambda i, *_: (i, 0, 0)),
            pl.BlockSpec((_RB, _ENT, _DIM), lambda i, *_: (i, 0, 0)),
            pl.BlockSpec((_CURR, _DIM), lambda i, *_: (0, 0)),
            pl.BlockSpec((_DIM, _DIM), lambda i, *_: (0, 0)),
            pl.BlockSpec((_DIM, _DIM), lambda i, *_: (0, 0)),
            pl.BlockSpec((_DIM, _DIM), lambda i, *_: (0, 0)),
        ],
        out_specs=pl.BlockSpec((_RB, _ENT, _DIM), lambda i, *_: (i, 0, 0)),
        scratch_shapes=[
            pltpu.VMEM((_CURR, _DIM), jnp.float32),
            pltpu.VMEM((_CURR + 8, _DIM), jnp.float32),
            pltpu.VMEM((_DIM, _DIM), jnp.bfloat16),
            pltpu.VMEM((_RF, _DIM), jnp.float32),
            pltpu.VMEM((_RF, _DIM), jnp.bfloat16),
            pltpu.VMEM((_RF, _DIM), jnp.float32),
        ],
    )
    return pl.pallas_call(
        _fused_body,
        grid_spec=grid_spec,
        out_shape=jax.ShapeDtypeStruct((_BATCH, _ENT, _DIM), jnp.float32),
        interpret=interpret,
    )(starts, order, rows, hiddens, keys, encoded_sents, U, V, W)


def kernel(encoded_sents, indices, hiddens, keys, U, V, W):
    return _run(encoded_sents, indices, hiddens, keys, U, V, W)


# speedup vs baseline: 1.0548x; 1.0548x over previous
import functools

import jax
import jax.numpy as jnp
from jax.experimental import pallas as pl
from jax.experimental.pallas import tpu as pltpu
from jax.experimental.pallas import tpu_sc as plsc

_BATCH = 1024
_ENT = 64
_DIM = 256
_CURR = 512
_RB = 64
_RF = _RB * _ENT


def _fused_body(idx_ref,
                h_ref, k_ref, s_ref, u_ref, v_ref, w_ref,
                o_ref,
                sw_ref, sb_ref, uvb_ref, acc_ref, hkb_ref, m_ref,
                starts_ref, cur_ref, order_ref, rows_ref):
    i = pl.program_id(0)

    @pl.when(i == 0)
    def _():
        uvb_ref[...] = (u_ref[...] + v_ref[...]).astype(jnp.bfloat16)
        sb_ref[0:_CURR, :] = s_ref[...]
        sb_ref[_CURR:, :] = jnp.zeros((8, _DIM), jnp.float32)
        sw_ref[...] = jnp.dot(s_ref[...], w_ref[...],
                              preferred_element_type=jnp.float32)

        def _zero(b, carry):
            starts_ref[b] = 0
            return carry
        jax.lax.fori_loop(0, _BATCH + 1, _zero, 0, unroll=False)

        def _hist(c, carry):
            starts_ref[idx_ref[c] + 1] += 1
            return carry
        jax.lax.fori_loop(0, _CURR, _hist, 0, unroll=False)

        def _prefix(b, carry):
            starts_ref[b + 1] += starts_ref[b]
            cur_ref[b] = starts_ref[b]
            return carry
        jax.lax.fori_loop(0, _BATCH, _prefix, 0, unroll=False)

        def _place(c, carry):
            v = idx_ref[c]
            p = cur_ref[v]
            order_ref[p] = c
            rows_ref[p] = v
            cur_ref[v] = p + 1
            return carry
        jax.lax.fori_loop(0, _CURR, _place, 0, unroll=False)

    hf = h_ref[...].reshape(_RF, _DIM)
    acc_ref[...] = hf
    hfb = hf.astype(jnp.bfloat16)
    hkb_ref[...] = hfb + k_ref[...].reshape(_RF, _DIM).astype(jnp.bfloat16)
    m_ref[...] = jnp.dot(hfb, uvb_ref[...],
                         preferred_element_type=jnp.float32)

    lo = starts_ref[i * _RB]
    hi = starts_ref[i * _RB + _RB]

    def seg_body(j, carry):
        c = order_ref[j]
        off = (rows_ref[j] - i * _RB) * _ENT
        s_c = sb_ref[pl.ds(c, 1), :]
        hk = hkb_ref[pl.ds(off, _ENT), :].astype(jnp.float32)
        gate = jax.nn.sigmoid(jax.lax.dot_general(
            hk, s_c, (((1,), (1,)), ((), ())),
            preferred_element_type=jnp.float32))
        sw = sw_ref[pl.ds((c % 8) * _ENT, _ENT), :]
        m = m_ref[pl.ds(off, _ENT), :]
        acc_ref[pl.ds(off, _ENT), :] += gate * jnp.maximum(m + sw, 0.0)
        return carry

    jax.lax.fori_loop(lo, hi, seg_body, 0, unroll=False)

    a = acc_ref[...]
    sq = jnp.sum(a * a, axis=1, keepdims=True)
    o_ref[...] = (a * jax.lax.rsqrt(jnp.maximum(sq, 1e-12))).reshape(
        _RB, _ENT, _DIM)


def _route_sc(indices):
    out_type = (jax.ShapeDtypeStruct((_BATCH + 1,), jnp.int32),
                jax.ShapeDtypeStruct((_CURR,), jnp.int32),
                jax.ShapeDtypeStruct((_CURR,), jnp.int32))
    mesh = plsc.ScalarSubcoreMesh(axis_name="core", num_cores=2)

    @functools.partial(
        pl.kernel, out_type=out_type, mesh=mesh,
        scratch_types=[pltpu.SMEM((_CURR,), jnp.int32),
                       pltpu.SMEM((_BATCH + 1,), jnp.int32),
                       pltpu.SMEM((_BATCH,), jnp.int32),
                       pltpu.SMEM((_CURR,), jnp.int32),
                       pltpu.SMEM((_CURR,), jnp.int32),
                       pltpu.SemaphoreType.DMA])
    def route_kernel(idx_hbm, starts_hbm, order_hbm, rows_hbm,
                     idx_s, starts_s, cur_s, order_s, rows_s, sem):
        @pl.when(jax.lax.axis_index("core") == 0)
        def _():
            pltpu.async_copy(idx_hbm, idx_s, sem).wait()

            @pl.loop(0, _BATCH + 1)
            def _(b):
                starts_s[b] = 0

            @pl.loop(0, _CURR)
            def _(c):
                starts_s[idx_s[c] + 1] += 1

            @pl.loop(0, _BATCH)
            def _(b):
                starts_s[b + 1] += starts_s[b]
                cur_s[b] = starts_s[b]

            @pl.loop(0, _CURR)
            def _(c):
                v = idx_s[c]
                p = cur_s[v]
                order_s[p] = c
                rows_s[p] = v
                cur_s[v] = p + 1

            pltpu.async_copy(starts_s, starts_hbm, sem).wait()
            pltpu.async_copy(order_s, order_hbm, sem).wait()
            pltpu.async_copy(rows_s, rows_hbm, sem).wait()

    return route_kernel(indices)


@functools.partial(jax.jit, static_argnames=("interpret",))
def _run(encoded_sents, indices, hiddens, keys, U, V, W, interpret=False):
    indices = indices.astype(jnp.int32)

    grid_spec = pltpu.PrefetchScalarGridSpec(
        num_scalar_prefetch=1,
        grid=(_BATCH // _RB,),
        in_specs=[
            pl.BlockSpec((_RB, _ENT, _DIM), lambda i, *_: (i, 0, 0)),
            pl.BlockSpec((_RB, _ENT, _DIM), lambda i, *_: (i, 0, 0)),
            pl.BlockSpec((_CURR, _DIM), lambda i, *_: (0, 0)),
            pl.BlockSpec((_DIM, _DIM), lambda i, *_: (0, 0)),
            pl.BlockSpec((_DIM, _DIM), lambda i, *_: (0, 0)),
            pl.BlockSpec((_DIM, _DIM), lambda i, *_: (0, 0)),
        ],
        out_specs=pl.BlockSpec((_RB, _ENT, _DIM), lambda i, *_: (i, 0, 0)),
        scratch_shapes=[
            pltpu.VMEM((_CURR, _DIM), jnp.float32),
            pltpu.VMEM((_CURR + 8, _DIM), jnp.float32),
            pltpu.VMEM((_DIM, _DIM), jnp.bfloat16),
            pltpu.VMEM((_RF, _DIM), jnp.float32),
            pltpu.VMEM((_RF, _DIM), jnp.bfloat16),
            pltpu.VMEM((_RF, _DIM), jnp.float32),
            pltpu.SMEM((_BATCH + 1,), jnp.int32),
            pltpu.SMEM((_BATCH,), jnp.int32),
            pltpu.SMEM((_CURR,), jnp.int32),
            pltpu.SMEM((_CURR,), jnp.int32),
        ],
    )
    return pl.pallas_call(
        _fused_body,
        grid_spec=grid_spec,
        out_shape=jax.ShapeDtypeStruct((_BATCH, _ENT, _DIM), jnp.float32),
        interpret=interpret,
    )(indices, hiddens, keys, encoded_sents, U, V, W)


def kernel(encoded_sents, indices, hiddens, keys, U, V, W):
    return _run(encoded_sents, indices, hiddens, keys, U, V, W)


# speedup vs baseline: 1.1476x; 1.0879x over previous
import functools

import jax
import jax.numpy as jnp
from jax.experimental import pallas as pl
from jax.experimental.pallas import tpu as pltpu
from jax.experimental.pallas import tpu_sc as plsc

_BATCH = 1024
_ENT = 64
_DIM = 256
_CURR = 512
_RB = 64
_RF = _RB * _ENT


def _fused_body(idx_ref,
                h_ref, k_ref, s_ref, u_ref, v_ref, w_ref,
                o_ref,
                sw_ref, sb_ref, uvb_ref, acc_ref, hkb_ref, m_ref,
                starts_ref, cur_ref, order_ref, rows_ref):
    i = pl.program_id(0)

    @pl.when(i == 0)
    def _():
        uvb_ref[...] = (u_ref[...] + v_ref[...]).astype(jnp.bfloat16)
        sb_ref[0:_CURR, :] = s_ref[...]
        sb_ref[_CURR:, :] = jnp.zeros((8, _DIM), jnp.float32)
        sw_ref[...] = jnp.dot(s_ref[...], w_ref[...],
                              preferred_element_type=jnp.float32)

        n_steps = _BATCH // _RB

        def _zero(b, carry):
            starts_ref[b] = 0
            return carry
        jax.lax.fori_loop(0, n_steps + 1, _zero, 0, unroll=False)

        def _hist(c, carry):
            starts_ref[idx_ref[c] // _RB + 1] += 1
            return carry
        jax.lax.fori_loop(0, _CURR, _hist, 0, unroll=False)

        def _prefix(b, carry):
            starts_ref[b + 1] += starts_ref[b]
            cur_ref[b] = starts_ref[b]
            return carry
        jax.lax.fori_loop(0, n_steps, _prefix, 0, unroll=False)

        def _place(c, carry):
            v = idx_ref[c]
            g = v // _RB
            p = cur_ref[g]
            order_ref[p] = c
            rows_ref[p] = v
            cur_ref[g] = p + 1
            return carry
        jax.lax.fori_loop(0, _CURR, _place, 0, unroll=False)

    hf = h_ref[...].reshape(_RF, _DIM)
    acc_ref[...] = hf
    hfb = hf.astype(jnp.bfloat16)
    hkb_ref[...] = hfb + k_ref[...].reshape(_RF, _DIM).astype(jnp.bfloat16)
    m_ref[...] = jnp.dot(hfb, uvb_ref[...],
                         preferred_element_type=jnp.float32)

    lo = starts_ref[i]
    hi = starts_ref[i + 1]

    def seg_body(j, carry):
        c = order_ref[j]
        off = (rows_ref[j] - i * _RB) * _ENT
        s_c = sb_ref[pl.ds(c, 1), :]
        hk = hkb_ref[pl.ds(off, _ENT), :].astype(jnp.float32)
        gate = jax.nn.sigmoid(jax.lax.dot_general(
            hk, s_c, (((1,), (1,)), ((), ())),
            preferred_element_type=jnp.float32))
        sw = sw_ref[pl.ds((c % 8) * _ENT, _ENT), :]
        m = m_ref[pl.ds(off, _ENT), :]
        acc_ref[pl.ds(off, _ENT), :] += gate * jnp.maximum(m + sw, 0.0)
        return carry

    jax.lax.fori_loop(lo, hi, seg_body, 0, unroll=False)

    a = acc_ref[...]
    sq = jnp.sum(a * a, axis=1, keepdims=True)
    o_ref[...] = (a * jax.lax.rsqrt(jnp.maximum(sq, 1e-12))).reshape(
        _RB, _ENT, _DIM)


def _route_sc(indices):
    out_type = (jax.ShapeDtypeStruct((_BATCH + 1,), jnp.int32),
                jax.ShapeDtypeStruct((_CURR,), jnp.int32),
                jax.ShapeDtypeStruct((_CURR,), jnp.int32))
    mesh = plsc.ScalarSubcoreMesh(axis_name="core", num_cores=2)

    @functools.partial(
        pl.kernel, out_type=out_type, mesh=mesh,
        scratch_types=[pltpu.SMEM((_CURR,), jnp.int32),
                       pltpu.SMEM((_BATCH + 1,), jnp.int32),
                       pltpu.SMEM((_BATCH,), jnp.int32),
                       pltpu.SMEM((_CURR,), jnp.int32),
                       pltpu.SMEM((_CURR,), jnp.int32),
                       pltpu.SemaphoreType.DMA])
    def route_kernel(idx_hbm, starts_hbm, order_hbm, rows_hbm,
                     idx_s, starts_s, cur_s, order_s, rows_s, sem):
        @pl.when(jax.lax.axis_index("core") == 0)
        def _():
            pltpu.async_copy(idx_hbm, idx_s, sem).wait()

            @pl.loop(0, _BATCH + 1)
            def _(b):
                starts_s[b] = 0

            @pl.loop(0, _CURR)
            def _(c):
                starts_s[idx_s[c] + 1] += 1

            @pl.loop(0, _BATCH)
            def _(b):
                starts_s[b + 1] += starts_s[b]
                cur_s[b] = starts_s[b]

            @pl.loop(0, _CURR)
            def _(c):
                v = idx_s[c]
                p = cur_s[v]
                order_s[p] = c
                rows_s[p] = v
                cur_s[v] = p + 1

            pltpu.async_copy(starts_s, starts_hbm, sem).wait()
            pltpu.async_copy(order_s, order_hbm, sem).wait()
            pltpu.async_copy(rows_s, rows_hbm, sem).wait()

    return route_kernel(indices)


@functools.partial(jax.jit, static_argnames=("interpret",))
def _run(encoded_sents, indices, hiddens, keys, U, V, W, interpret=False):
    indices = indices.astype(jnp.int32)

    grid_spec = pltpu.PrefetchScalarGridSpec(
        num_scalar_prefetch=1,
        grid=(_BATCH // _RB,),
        in_specs=[
            pl.BlockSpec((_RB, _ENT, _DIM), lambda i, *_: (i, 0, 0)),
            pl.BlockSpec((_RB, _ENT, _DIM), lambda i, *_: (i, 0, 0)),
            pl.BlockSpec((_CURR, _DIM), lambda i, *_: (0, 0)),
            pl.BlockSpec((_DIM, _DIM), lambda i, *_: (0, 0)),
            pl.BlockSpec((_DIM, _DIM), lambda i, *_: (0, 0)),
            pl.BlockSpec((_DIM, _DIM), lambda i, *_: (0, 0)),
        ],
        out_specs=pl.BlockSpec((_RB, _ENT, _DIM), lambda i, *_: (i, 0, 0)),
        scratch_shapes=[
            pltpu.VMEM((_CURR, _DIM), jnp.float32),
            pltpu.VMEM((_CURR + 8, _DIM), jnp.float32),
            pltpu.VMEM((_DIM, _DIM), jnp.bfloat16),
            pltpu.VMEM((_RF, _DIM), jnp.float32),
            pltpu.VMEM((_RF, _DIM), jnp.bfloat16),
            pltpu.VMEM((_RF, _DIM), jnp.float32),
            pltpu.SMEM((_BATCH // _RB + 1,), jnp.int32),
            pltpu.SMEM((_BATCH // _RB,), jnp.int32),
            pltpu.SMEM((_CURR,), jnp.int32),
            pltpu.SMEM((_CURR,), jnp.int32),
        ],
    )
    return pl.pallas_call(
        _fused_body,
        grid_spec=grid_spec,
        out_shape=jax.ShapeDtypeStruct((_BATCH, _ENT, _DIM), jnp.float32),
        interpret=interpret,
    )(indices, hiddens, keys, encoded_sents, U, V, W)


def kernel(encoded_sents, indices, hiddens, keys, U, V, W):
    return _run(encoded_sents, indices, hiddens, keys, U, V, W)


# speedup vs baseline: 1.1521x; 1.0040x over previous
import functools

import jax
import jax.numpy as jnp
from jax.experimental import pallas as pl
from jax.experimental.pallas import tpu as pltpu

_BATCH = 1024
_ENT = 64
_DIM = 256
_CURR = 512
_RB = 64
_RF = _RB * _ENT


def _fused_body(idx_ref,
                h_ref, k_ref, s_ref, u_ref, v_ref, w_ref,
                o_ref,
                sw_ref, sb_ref, uvb_ref, acc_ref, hkb_ref, m_ref,
                starts_ref, cur_ref, order_ref, rows_ref):
    i = pl.program_id(0)

    @pl.when(i == 0)
    def _():
        uvb_ref[...] = (u_ref[...] + v_ref[...]).astype(jnp.bfloat16)
        sb_ref[0:_CURR, :] = s_ref[...]
        sb_ref[_CURR:, :] = jnp.zeros((8, _DIM), jnp.float32)
        sw_ref[...] = jnp.dot(s_ref[...], w_ref[...],
                              preferred_element_type=jnp.float32)

        n_steps = _BATCH // _RB

        def _zero(b, carry):
            starts_ref[b] = 0
            return carry
        jax.lax.fori_loop(0, n_steps + 1, _zero, 0, unroll=False)

        def _hist(c, carry):
            starts_ref[idx_ref[c] // _RB + 1] += 1
            return carry
        jax.lax.fori_loop(0, _CURR, _hist, 0, unroll=False)

        def _prefix(b, carry):
            starts_ref[b + 1] += starts_ref[b]
            cur_ref[b] = starts_ref[b]
            return carry
        jax.lax.fori_loop(0, n_steps, _prefix, 0, unroll=False)

        def _place(c, carry):
            v = idx_ref[c]
            g = v // _RB
            p = cur_ref[g]
            order_ref[p] = c
            rows_ref[p] = v
            cur_ref[g] = p + 1
            return carry
        jax.lax.fori_loop(0, _CURR, _place, 0, unroll=False)

    hf = h_ref[...].reshape(_RF, _DIM)
    acc_ref[...] = hf
    hfb = hf.astype(jnp.bfloat16)
    hkb_ref[...] = hfb + k_ref[...].reshape(_RF, _DIM).astype(jnp.bfloat16)
    m_ref[...] = jnp.dot(hfb, uvb_ref[...],
                         preferred_element_type=jnp.float32)

    lo = starts_ref[i]
    hi = starts_ref[i + 1]

    def seg_body(j, carry):
        c = order_ref[j]
        off = (rows_ref[j] - i * _RB) * _ENT
        s_c = sb_ref[pl.ds(c, 1), :]
        hk = hkb_ref[pl.ds(off, _ENT), :].astype(jnp.float32)
        gate = jax.nn.sigmoid(jax.lax.dot_general(
            hk, s_c, (((1,), (1,)), ((), ())),
            preferred_element_type=jnp.float32))
        sw = sw_ref[pl.ds((c % 8) * _ENT, _ENT), :]
        m = m_ref[pl.ds(off, _ENT), :]
        acc_ref[pl.ds(off, _ENT), :] += gate * jnp.maximum(m + sw, 0.0)
        return carry

    jax.lax.fori_loop(lo, hi, seg_body, 0, unroll=False)

    a = acc_ref[...]
    sq = jnp.sum(a * a, axis=1, keepdims=True)
    o_ref[...] = (a * jax.lax.rsqrt(jnp.maximum(sq, 1e-12))).reshape(
        _RB, _ENT, _DIM)


@functools.partial(jax.jit, static_argnames=("interpret",))
def _run(encoded_sents, indices, hiddens, keys, U, V, W, interpret=False):
    indices = indices.astype(jnp.int32)

    grid_spec = pltpu.PrefetchScalarGridSpec(
        num_scalar_prefetch=1,
        grid=(_BATCH // _RB,),
        in_specs=[
            pl.BlockSpec((_RB, _ENT, _DIM), lambda i, *_: (i, 0, 0)),
            pl.BlockSpec((_RB, _ENT, _DIM), lambda i, *_: (i, 0, 0)),
            pl.BlockSpec((_CURR, _DIM), lambda i, *_: (0, 0)),
            pl.BlockSpec((_DIM, _DIM), lambda i, *_: (0, 0)),
            pl.BlockSpec((_DIM, _DIM), lambda i, *_: (0, 0)),
            pl.BlockSpec((_DIM, _DIM), lambda i, *_: (0, 0)),
        ],
        out_specs=pl.BlockSpec((_RB, _ENT, _DIM), lambda i, *_: (i, 0, 0)),
        scratch_shapes=[
            pltpu.VMEM((_CURR, _DIM), jnp.float32),
            pltpu.VMEM((_CURR + 8, _DIM), jnp.float32),
            pltpu.VMEM((_DIM, _DIM), jnp.bfloat16),
            pltpu.VMEM((_RF, _DIM), jnp.float32),
            pltpu.VMEM((_RF, _DIM), jnp.bfloat16),
            pltpu.VMEM((_RF, _DIM), jnp.float32),
            pltpu.SMEM((_BATCH // _RB + 1,), jnp.int32),
            pltpu.SMEM((_BATCH // _RB,), jnp.int32),
            pltpu.SMEM((_CURR,), jnp.int32),
            pltpu.SMEM((_CURR,), jnp.int32),
        ],
    )
    return pl.pallas_call(
        _fused_body,
        grid_spec=grid_spec,
        out_shape=jax.ShapeDtypeStruct((_BATCH, _ENT, _DIM), jnp.float32),
        interpret=interpret,
    )(indices, hiddens, keys, encoded_sents, U, V, W)


def kernel(encoded_sents, indices, hiddens, keys, U, V, W):
    return _run(encoded_sents, indices, hiddens, keys, U, V, W)
